# Initial kernel scaffold; baseline (speedup 1.0000x reference)
import jax
import jax.numpy as jnp
from jax.experimental import pallas as pl


def kernel(x, features, geometry, edge_index, W1, b1, W2, b2, fc1_W, fc1_b, fc2_W, fc2_b, fc3_W, fc3_b):
    raise NotImplementedError("write your pallas kernel here")


# speedup vs baseline: 9.5333x; 9.5333x over previous
import functools

import jax
import jax.numpy as jnp
import numpy as np
from jax import lax
from jax.experimental import pallas as pl
from jax.experimental.pallas import tpu as pltpu
from jax.experimental.pallas import tpu_sc as plsc

N_NODES = 50000
N_EDGES = 1600000
MAX_RADIUS = 3.0
N_BASIS = 3

NC = 2
NS = 16
NW = NC * NS

EB = 8192
NBLK = 196
E_PAD = EB * NBLK
ROWS_TOTAL = E_PAD // 128
ROWS_PW = ROWS_TOTAL // NW
R_G = 8
CH_G = ROWS_PW // R_G
CE = R_G * 128
R_S = 8
CH_S = ROWS_PW // R_S
N_TAB = 51200
N_SLICE = N_TAB // NS
DPL = 24
STEP = MAX_RADIUS / (N_BASIS - 1)
INV_NORM = float(1.0 / np.sqrt(N_EDGES / N_NODES))

_mesh = plsc.VectorSubcoreMesh(core_axis_name="c", subcore_axis_name="s")


@functools.partial(
    pl.kernel,
    out_type=(
        jax.ShapeDtypeStruct((E_PAD,), jnp.float32),
        jax.ShapeDtypeStruct((DPL, E_PAD), jnp.float32),
    ),
    mesh=_mesh,
    scratch_types=[
        pltpu.VMEM((R_G, 128), jnp.int32),
        pltpu.VMEM((R_G, 128), jnp.int32),
        pltpu.VMEM((CE,), jnp.float32),
        pltpu.VMEM((CE,), jnp.float32),
        pltpu.VMEM((CE,), jnp.float32),
        pltpu.VMEM((CE,), jnp.float32),
        pltpu.VMEM((CE,), jnp.float32),
        pltpu.VMEM((CE,), jnp.float32),
        pltpu.VMEM((CE,), jnp.float32),
    ] + [pltpu.VMEM((CE,), jnp.float32) for _ in range(DPL)]
      + [pltpu.VMEM_SHARED((N_TAB,), jnp.float32) for _ in range(3)]
      + [pltpu.VMEM_SHARED((N_TAB,), jnp.float32) for _ in range(DPL)]
      + [pltpu.SemaphoreType.DMA],
)
def _sc_gather(src_hbm, dst_hbm, gx_hbm, gy_hbm, gz_hbm, ft_hbm,
               d2_hbm, fT_hbm,
               sidx, didx, gxs, gys, gzs, gxd, gyd, gzd, d2b, *rest):
    fpl = rest[:DPL]
    gx_sh, gy_sh, gz_sh = rest[DPL:DPL + 3]
    feat_sh = rest[DPL + 3:DPL + 3 + DPL]
    sem = rest[-1]
    sid = lax.axis_index("s")
    wid = sid * NC + lax.axis_index("c")
    row0 = wid * ROWS_PW

    t0 = sid * N_SLICE
    pltpu.sync_copy(gx_hbm.at[pl.ds(t0, N_SLICE)], gx_sh.at[pl.ds(t0, N_SLICE)])
    pltpu.sync_copy(gy_hbm.at[pl.ds(t0, N_SLICE)], gy_sh.at[pl.ds(t0, N_SLICE)])
    pltpu.sync_copy(gz_hbm.at[pl.ds(t0, N_SLICE)], gz_sh.at[pl.ds(t0, N_SLICE)])
    for d in range(DPL):
        pltpu.sync_copy(ft_hbm.at[d, pl.ds(t0, N_SLICE)],
                        feat_sh[d].at[pl.ds(t0, N_SLICE)])
    plsc.subcore_barrier()

    def chunk(t, carry):
        roff = row0 + t * R_G
        eoff = roff * 128
        pltpu.sync_copy(src_hbm.at[pl.ds(roff, R_G)], sidx)
        pltpu.sync_copy(dst_hbm.at[pl.ds(roff, R_G)], didx)

        def jstep(j, jcarry):
            o = j * 128
            descs = [
                pltpu.async_copy(gx_sh.at[sidx.at[j]], gxs.at[pl.ds(o, 128)], sem),
                pltpu.async_copy(gy_sh.at[sidx.at[j]], gys.at[pl.ds(o, 128)], sem),
                pltpu.async_copy(gz_sh.at[sidx.at[j]], gzs.at[pl.ds(o, 128)], sem),
                pltpu.async_copy(gx_sh.at[didx.at[j]], gxd.at[pl.ds(o, 128)], sem),
                pltpu.async_copy(gy_sh.at[didx.at[j]], gyd.at[pl.ds(o, 128)], sem),
                pltpu.async_copy(gz_sh.at[didx.at[j]], gzd.at[pl.ds(o, 128)], sem),
            ]
            for d in range(DPL):
                descs.append(pltpu.async_copy(
                    feat_sh[d].at[sidx.at[j]], fpl[d].at[pl.ds(o, 128)], sem))
            for dd in descs:
                dd.wait()
            return jcarry

        lax.fori_loop(0, R_G, jstep, 0)

        def vstep(l, vcarry):
            s = pl.ds(l * 16, 16)
            dx = gxs[s] - gxd[s]
            dy = gys[s] - gyd[s]
            dz = gzs[s] - gzd[s]
            d2b[s] = dx * dx + dy * dy + dz * dz
            return vcarry

        lax.fori_loop(0, CE // 16, vstep, 0)
        pltpu.sync_copy(d2b, d2_hbm.at[pl.ds(eoff, CE)])
        for d in range(DPL):
            pltpu.sync_copy(fpl[d], fT_hbm.at[d, pl.ds(eoff, CE)])
        return carry

    lax.fori_loop(0, CH_G, chunk, 0)


@functools.partial(
    pl.kernel,
    out_type=jax.ShapeDtypeStruct((NC, N_TAB), jnp.float32),
    mesh=_mesh,
    scratch_types=[
        pltpu.VMEM((R_S, 128), jnp.int32),
        pltpu.VMEM((R_S, 128), jnp.float32),
        pltpu.VMEM((N_SLICE,), jnp.float32),
        pltpu.VMEM_SHARED((N_TAB,), jnp.float32),
        pltpu.SemaphoreType.DMA,
    ],
)
def _sc_scatter(dst_hbm, msg_hbm, part_hbm, didx, mv, zbuf, acc, sem):
    cid = lax.axis_index("c")
    sid = lax.axis_index("s")
    wid = sid * NC + cid

    def zstep(i, carry):
        zbuf[pl.ds(i * 16, 16)] = jnp.zeros((16,), jnp.float32)
        return carry
    lax.fori_loop(0, N_SLICE // 16, zstep, 0)
    pltpu.sync_copy(zbuf, acc.at[pl.ds(sid * N_SLICE, N_SLICE)])
    plsc.subcore_barrier()

    row0 = wid * ROWS_PW

    def chunk(t, carry):
        roff = row0 + t * R_S
        pltpu.sync_copy(dst_hbm.at[pl.ds(roff, R_S)], didx)
        pltpu.sync_copy(msg_hbm.at[pl.ds(roff, R_S)], mv)
        for j in range(R_S):
            pltpu.sync_copy(mv.at[j], acc.at[didx.at[j]], add=True)
        return carry

    lax.fori_loop(0, CH_S, chunk, 0)
    plsc.subcore_barrier()
    pltpu.sync_copy(acc.at[pl.ds(sid * N_SLICE, N_SLICE)],
                    part_hbm.at[cid, pl.ds(sid * N_SLICE, N_SLICE)])


def _tc_edge_body(d2r, ftr, w1t, b1c, w2t, b2c, out):
    r = jnp.sqrt(d2r[...][0] + 1e-9)
    c8 = lax.broadcasted_iota(jnp.int32, (8, 1), 0).astype(jnp.float32) * STEP
    d = (r - c8) * (1.0 / STEP)
    cosd = jnp.cos((0.5 * np.pi) * d)
    basis = jnp.where(jnp.abs(d) < 1.0, cosd * cosd, 0.0)
    h = jnp.dot(w1t[...], basis, preferred_element_type=jnp.float32) + b1c[...]
    h = jnp.maximum(h, 0.0)
    w = jnp.dot(w2t[...], h, preferred_element_type=jnp.float32) + b2c[...]
    msg = jnp.sum(w[0:DPL, :] * ftr[...], axis=0, keepdims=True)
    eglob = lax.broadcasted_iota(jnp.int32, (1, EB), 1) + pl.program_id(0) * EB
    res = jnp.where(eglob < N_EDGES, msg * INV_NORM, 0.0)
    out[...] = res.reshape(1, 1, EB)


_tc_edge = pl.pallas_call(
    _tc_edge_body,
    grid=(NBLK,),
    in_specs=[
        pl.BlockSpec((1, 1, EB), lambda i: (i, 0, 0)),
        pl.BlockSpec((DPL, EB), lambda i: (0, i)),
        pl.BlockSpec((128, 8), lambda i: (0, 0)),
        pl.BlockSpec((128, 1), lambda i: (0, 0)),
        pl.BlockSpec((32, 128), lambda i: (0, 0)),
        pl.BlockSpec((32, 1), lambda i: (0, 0)),
    ],
    out_specs=pl.BlockSpec((1, 1, EB), lambda i: (i, 0, 0)),
    out_shape=jax.ShapeDtypeStruct((NBLK, 1, EB), jnp.float32),
)


def _tc_readout_body(parts, w1, b1, w2, b2, w3, b3, out):
    feat = parts[0:1, :] + parts[1:2, :]
    h1 = jnp.dot(feat, w1[...], preferred_element_type=jnp.float32) + b1[...]
    h1 = jnp.maximum(h1, 0.0)
    h2 = jnp.dot(h1, w2[...], preferred_element_type=jnp.float32) + b2[...]
    h2 = jnp.maximum(h2, 0.0)
    out[...] = jnp.dot(h2, w3[...], preferred_element_type=jnp.float32) + b3[...]


_tc_readout = pl.pallas_call(
    _tc_readout_body,
    out_shape=jax.ShapeDtypeStruct((1, 8), jnp.float32),
)


def kernel(x, features, geometry, edge_index, W1, b1, W2, b2,
           fc1_W, fc1_b, fc2_W, fc2_b, fc3_W, fc3_b):
    n = features.shape[0]
    e = edge_index.shape[1]

    src = jnp.pad(edge_index[0], (0, E_PAD - e)).reshape(ROWS_TOTAL, 128)
    dst = jnp.pad(edge_index[1], (0, E_PAD - e)).reshape(ROWS_TOTAL, 128)
    gx = jnp.pad(geometry[:, 0], (0, N_TAB - n))
    gy = jnp.pad(geometry[:, 1], (0, N_TAB - n))
    gz = jnp.pad(geometry[:, 2], (0, N_TAB - n))
    featT = jnp.pad(features.T, ((0, DPL - features.shape[1]), (0, N_TAB - n)))

    w1t = jnp.pad(W1, ((0, 8 - W1.shape[0]), (0, 128 - W1.shape[1]))).T
    b1c = jnp.pad(b1, (0, 128 - b1.shape[0])).reshape(128, 1)
    w2t = jnp.pad(W2, ((0, 128 - W2.shape[0]), (0, 32 - W2.shape[1]))).T
    b2c = jnp.pad(b2, (0, 32 - b2.shape[0])).reshape(32, 1)

    fc1_wp = jnp.pad(fc1_W, ((0, N_TAB - n), (0, 32 - fc1_W.shape[1])))
    fc1_bp = jnp.pad(fc1_b, (0, 32 - fc1_b.shape[0])).reshape(1, 32)
    fc2_wp = jnp.pad(fc2_W, ((0, 32 - fc2_W.shape[0]), (0, 16 - fc2_W.shape[1])))
    fc2_bp = jnp.pad(fc2_b, (0, 16 - fc2_b.shape[0])).reshape(1, 16)
    fc3_wp = jnp.pad(fc3_W, ((0, 16 - fc3_W.shape[0]), (0, 8 - fc3_W.shape[1])))
    fc3_bp = jnp.pad(fc3_b, (0, 8 - fc3_b.shape[0])).reshape(1, 8)

    d2, fT = _sc_gather(src, dst, gx, gy, gz, featT)
    msg = _tc_edge(d2.reshape(NBLK, 1, EB), fT, w1t, b1c, w2t, b2c)
    parts = _sc_scatter(dst, msg.reshape(ROWS_TOTAL, 128))
    out = _tc_readout(parts, fc1_wp, fc1_bp, fc2_wp, fc2_bp, fc3_wp, fc3_bp)
    return out[0, 0:1]


# speedup vs baseline: 10.2671x; 1.0770x over previous
import functools

import jax
import jax.numpy as jnp
import numpy as np
from jax import lax
from jax.experimental import pallas as pl
from jax.experimental.pallas import tpu as pltpu
from jax.experimental.pallas import tpu_sc as plsc

N_NODES = 50000
N_EDGES = 1600000
MAX_RADIUS = 3.0
N_BASIS = 3

NC = 2
NS = 16
NW = NC * NS

EB = 8192
NBLK = 196
E_PAD = EB * NBLK
E_PW = E_PAD // NW
CE = 1024
CH_G = E_PW // CE
CS = 3584
CH_S = E_PW // CS
N_TAB = 51200
N_SLICE = N_TAB // NS
DPL = 24
STEP = MAX_RADIUS / (N_BASIS - 1)
INV_NORM = float(1.0 / np.sqrt(N_EDGES / N_NODES))

_mesh = plsc.VectorSubcoreMesh(core_axis_name="c", subcore_axis_name="s")


@functools.partial(
    pl.kernel,
    out_type=(
        jax.ShapeDtypeStruct((E_PAD,), jnp.float32),
        jax.ShapeDtypeStruct((DPL, E_PAD), jnp.float32),
    ),
    mesh=_mesh,
    scratch_types=[
        pltpu.VMEM((CE,), jnp.int32),
        pltpu.VMEM((CE,), jnp.int32),
        pltpu.VMEM((CE,), jnp.float32),
        pltpu.VMEM((CE,), jnp.float32),
        pltpu.VMEM((CE,), jnp.float32),
        pltpu.VMEM((CE,), jnp.float32),
        pltpu.VMEM((CE,), jnp.float32),
        pltpu.VMEM((CE,), jnp.float32),
        pltpu.VMEM((CE,), jnp.float32),
    ] + [pltpu.VMEM((CE,), jnp.float32) for _ in range(DPL)]
      + [pltpu.VMEM_SHARED((N_TAB,), jnp.float32) for _ in range(3)]
      + [pltpu.VMEM_SHARED((N_TAB,), jnp.float32) for _ in range(DPL)]
      + [pltpu.SemaphoreType.DMA],
)
def _sc_gather(src_hbm, dst_hbm, gx_hbm, gy_hbm, gz_hbm, ft_hbm,
               d2_hbm, fT_hbm,
               sidx, didx, gxs, gys, gzs, gxd, gyd, gzd, d2b, *rest):
    fpl = rest[:DPL]
    gx_sh, gy_sh, gz_sh = rest[DPL:DPL + 3]
    feat_sh = rest[DPL + 3:DPL + 3 + DPL]
    sem = rest[-1]
    sid = lax.axis_index("s")
    wid = sid * NC + lax.axis_index("c")
    e0 = wid * E_PW

    t0 = sid * N_SLICE
    pltpu.sync_copy(gx_hbm.at[pl.ds(t0, N_SLICE)], gx_sh.at[pl.ds(t0, N_SLICE)])
    pltpu.sync_copy(gy_hbm.at[pl.ds(t0, N_SLICE)], gy_sh.at[pl.ds(t0, N_SLICE)])
    pltpu.sync_copy(gz_hbm.at[pl.ds(t0, N_SLICE)], gz_sh.at[pl.ds(t0, N_SLICE)])
    for d in range(DPL):
        pltpu.sync_copy(ft_hbm.at[d, pl.ds(t0, N_SLICE)],
                        feat_sh[d].at[pl.ds(t0, N_SLICE)])
    plsc.subcore_barrier()

    def chunk(t, carry):
        eoff = e0 + t * CE
        pltpu.sync_copy(src_hbm.at[pl.ds(eoff, CE)], sidx)
        pltpu.sync_copy(dst_hbm.at[pl.ds(eoff, CE)], didx)
        descs = [
            pltpu.async_copy(gx_sh.at[sidx], gxs, sem),
            pltpu.async_copy(gy_sh.at[sidx], gys, sem),
            pltpu.async_copy(gz_sh.at[sidx], gzs, sem),
            pltpu.async_copy(gx_sh.at[didx], gxd, sem),
            pltpu.async_copy(gy_sh.at[didx], gyd, sem),
            pltpu.async_copy(gz_sh.at[didx], gzd, sem),
        ]
        for d in range(DPL):
            descs.append(pltpu.async_copy(feat_sh[d].at[sidx], fpl[d], sem))
        for dd in descs:
            dd.wait()

        def vstep(l, vcarry):
            s = pl.ds(l * 16, 16)
            dx = gxs[s] - gxd[s]
            dy = gys[s] - gyd[s]
            dz = gzs[s] - gzd[s]
            d2b[s] = dx * dx + dy * dy + dz * dz
            return vcarry

        lax.fori_loop(0, CE // 16, vstep, 0)
        pltpu.sync_copy(d2b, d2_hbm.at[pl.ds(eoff, CE)])
        for d in range(DPL):
            pltpu.sync_copy(fpl[d], fT_hbm.at[d, pl.ds(eoff, CE)])
        return carry

    lax.fori_loop(0, CH_G, chunk, 0)


@functools.partial(
    pl.kernel,
    out_type=jax.ShapeDtypeStruct((NC, N_TAB), jnp.float32),
    mesh=_mesh,
    scratch_types=[
        pltpu.VMEM((CS,), jnp.int32),
        pltpu.VMEM((CS,), jnp.float32),
        pltpu.VMEM((N_SLICE,), jnp.float32),
        pltpu.VMEM_SHARED((N_TAB,), jnp.float32),
        pltpu.SemaphoreType.DMA,
    ],
)
def _sc_scatter(dst_hbm, msg_hbm, part_hbm, didx, mv, zbuf, acc, sem):
    cid = lax.axis_index("c")
    sid = lax.axis_index("s")
    wid = sid * NC + cid

    def zstep(i, carry):
        zbuf[pl.ds(i * 16, 16)] = jnp.zeros((16,), jnp.float32)
        return carry
    lax.fori_loop(0, N_SLICE // 16, zstep, 0)
    pltpu.sync_copy(zbuf, acc.at[pl.ds(sid * N_SLICE, N_SLICE)])
    plsc.subcore_barrier()

    e0 = wid * E_PW

    def chunk(t, carry):
        eoff = e0 + t * CS
        pltpu.sync_copy(dst_hbm.at[pl.ds(eoff, CS)], didx)
        pltpu.sync_copy(msg_hbm.at[pl.ds(eoff, CS)], mv)
        pltpu.sync_copy(mv, acc.at[didx], add=True)
        return carry

    lax.fori_loop(0, CH_S, chunk, 0)
    plsc.subcore_barrier()
    pltpu.sync_copy(acc.at[pl.ds(sid * N_SLICE, N_SLICE)],
                    part_hbm.at[cid, pl.ds(sid * N_SLICE, N_SLICE)])


def _tc_edge_body(d2r, ftr, w1t, b1c, w2t, b2c, out):
    r = jnp.sqrt(d2r[...][0] + 1e-9)
    c8 = lax.broadcasted_iota(jnp.int32, (8, 1), 0).astype(jnp.float32) * STEP
    d = (r - c8) * (1.0 / STEP)
    cosd = jnp.cos((0.5 * np.pi) * d)
    basis = jnp.where(jnp.abs(d) < 1.0, cosd * cosd, 0.0)
    h = jnp.dot(w1t[...], basis, preferred_element_type=jnp.float32) + b1c[...]
    h = jnp.maximum(h, 0.0)
    w = jnp.dot(w2t[...], h, preferred_element_type=jnp.float32) + b2c[...]
    msg = jnp.sum(w[0:DPL, :] * ftr[...], axis=0, keepdims=True)
    eglob = lax.broadcasted_iota(jnp.int32, (1, EB), 1) + pl.program_id(0) * EB
    res = jnp.where(eglob < N_EDGES, msg * INV_NORM, 0.0)
    out[...] = res.reshape(1, 1, EB)


_tc_edge = pl.pallas_call(
    _tc_edge_body,
    grid=(NBLK,),
    in_specs=[
        pl.BlockSpec((1, 1, EB), lambda i: (i, 0, 0)),
        pl.BlockSpec((DPL, EB), lambda i: (0, i)),
        pl.BlockSpec((128, 8), lambda i: (0, 0)),
        pl.BlockSpec((128, 1), lambda i: (0, 0)),
        pl.BlockSpec((32, 128), lambda i: (0, 0)),
        pl.BlockSpec((32, 1), lambda i: (0, 0)),
    ],
    out_specs=pl.BlockSpec((1, 1, EB), lambda i: (i, 0, 0)),
    out_shape=jax.ShapeDtypeStruct((NBLK, 1, EB), jnp.float32),
)


def _tc_readout_body(parts, w1, b1, w2, b2, w3, b3, out):
    feat = parts[0:1, :] + parts[1:2, :]
    h1 = jnp.dot(feat, w1[...], preferred_element_type=jnp.float32) + b1[...]
    h1 = jnp.maximum(h1, 0.0)
    h2 = jnp.dot(h1, w2[...], preferred_element_type=jnp.float32) + b2[...]
    h2 = jnp.maximum(h2, 0.0)
    out[...] = jnp.dot(h2, w3[...], preferred_element_type=jnp.float32) + b3[...]


_tc_readout = pl.pallas_call(
    _tc_readout_body,
    out_shape=jax.ShapeDtypeStruct((1, 8), jnp.float32),
)


def kernel(x, features, geometry, edge_index, W1, b1, W2, b2,
           fc1_W, fc1_b, fc2_W, fc2_b, fc3_W, fc3_b):
    n = features.shape[0]
    e = edge_index.shape[1]

    src = jnp.pad(edge_index[0], (0, E_PAD - e))
    dst = jnp.pad(edge_index[1], (0, E_PAD - e))
    gx = jnp.pad(geometry[:, 0], (0, N_TAB - n))
    gy = jnp.pad(geometry[:, 1], (0, N_TAB - n))
    gz = jnp.pad(geometry[:, 2], (0, N_TAB - n))
    featT = jnp.pad(features.T, ((0, DPL - features.shape[1]), (0, N_TAB - n)))

    w1t = jnp.pad(W1, ((0, 8 - W1.shape[0]), (0, 128 - W1.shape[1]))).T
    b1c = jnp.pad(b1, (0, 128 - b1.shape[0])).reshape(128, 1)
    w2t = jnp.pad(W2, ((0, 128 - W2.shape[0]), (0, 32 - W2.shape[1]))).T
    b2c = jnp.pad(b2, (0, 32 - b2.shape[0])).reshape(32, 1)

    fc1_wp = jnp.pad(fc1_W, ((0, N_TAB - n), (0, 32 - fc1_W.shape[1])))
    fc1_bp = jnp.pad(fc1_b, (0, 32 - fc1_b.shape[0])).reshape(1, 32)
    fc2_wp = jnp.pad(fc2_W, ((0, 32 - fc2_W.shape[0]), (0, 16 - fc2_W.shape[1])))
    fc2_bp = jnp.pad(fc2_b, (0, 16 - fc2_b.shape[0])).reshape(1, 16)
    fc3_wp = jnp.pad(fc3_W, ((0, 16 - fc3_W.shape[0]), (0, 8 - fc3_W.shape[1])))
    fc3_bp = jnp.pad(fc3_b, (0, 8 - fc3_b.shape[0])).reshape(1, 8)

    d2, fT = _sc_gather(src, dst, gx, gy, gz, featT)
    msg = _tc_edge(d2.reshape(NBLK, 1, EB), fT, w1t, b1c, w2t, b2c)
    parts = _sc_scatter(dst, msg.reshape(E_PAD))
    out = _tc_readout(parts, fc1_wp, fc1_bp, fc2_wp, fc2_bp, fc3_wp, fc3_bp)
    return out[0, 0:1]


# speedup vs baseline: 11.9048x; 1.1595x over previous
import functools

import jax
import jax.numpy as jnp
import numpy as np
from jax import lax
from jax.experimental import pallas as pl
from jax.experimental.pallas import tpu as pltpu
from jax.experimental.pallas import tpu_sc as plsc

N_NODES = 50000
N_EDGES = 1600000
MAX_RADIUS = 3.0
N_BASIS = 3

NC = 2
NS = 16
NW = NC * NS

EB = 8192
NBLK = 196
E_PAD = EB * NBLK
K_CH = 4
EK = E_PAD // K_CH
NBLK_K = NBLK // K_CH
E_PW = EK // NW
CE = 896
CH_G = E_PW // CE
CS = 1792
CH_S = E_PW // CS
N_TAB = 51200
N_SLICE = N_TAB // NS
DPL = 24
STEP = MAX_RADIUS / (N_BASIS - 1)
INV_NORM = float(1.0 / np.sqrt(N_EDGES / N_NODES))

_mesh = plsc.VectorSubcoreMesh(core_axis_name="c", subcore_axis_name="s")


@functools.partial(
    pl.kernel,
    out_type=(
        jax.ShapeDtypeStruct((EK,), jnp.float32),
        jax.ShapeDtypeStruct((DPL, EK), jnp.float32),
    ),
    mesh=_mesh,
    scratch_types=[
        pltpu.VMEM((CE,), jnp.int32),
        pltpu.VMEM((CE,), jnp.int32),
        pltpu.VMEM((CE,), jnp.float32),
        pltpu.VMEM((CE,), jnp.float32),
        pltpu.VMEM((CE,), jnp.float32),
        pltpu.VMEM((CE,), jnp.float32),
        pltpu.VMEM((CE,), jnp.float32),
        pltpu.VMEM((CE,), jnp.float32),
        pltpu.VMEM((CE,), jnp.float32),
    ] + [pltpu.VMEM((CE,), jnp.float32) for _ in range(DPL)]
      + [pltpu.VMEM_SHARED((N_TAB,), jnp.float32) for _ in range(3)]
      + [pltpu.VMEM_SHARED((N_TAB,), jnp.float32) for _ in range(DPL)]
      + [pltpu.SemaphoreType.DMA],
)
def _sc_gather(src_hbm, dst_hbm, gx_hbm, gy_hbm, gz_hbm, ft_hbm,
               d2_hbm, fT_hbm,
               sidx, didx, gxs, gys, gzs, gxd, gyd, gzd, d2b, *rest):
    fpl = rest[:DPL]
    gx_sh, gy_sh, gz_sh = rest[DPL:DPL + 3]
    feat_sh = rest[DPL + 3:DPL + 3 + DPL]
    sem = rest[-1]
    sid = lax.axis_index("s")
    wid = sid * NC + lax.axis_index("c")
    e0 = wid * E_PW

    t0 = sid * N_SLICE
    pltpu.sync_copy(gx_hbm.at[pl.ds(t0, N_SLICE)], gx_sh.at[pl.ds(t0, N_SLICE)])
    pltpu.sync_copy(gy_hbm.at[pl.ds(t0, N_SLICE)], gy_sh.at[pl.ds(t0, N_SLICE)])
    pltpu.sync_copy(gz_hbm.at[pl.ds(t0, N_SLICE)], gz_sh.at[pl.ds(t0, N_SLICE)])
    for d in range(DPL):
        pltpu.sync_copy(ft_hbm.at[d, pl.ds(t0, N_SLICE)],
                        feat_sh[d].at[pl.ds(t0, N_SLICE)])
    plsc.subcore_barrier()

    def chunk(t, carry):
        eoff = e0 + t * CE
        pltpu.sync_copy(src_hbm.at[pl.ds(eoff, CE)], sidx)
        pltpu.sync_copy(dst_hbm.at[pl.ds(eoff, CE)], didx)
        descs = [
            pltpu.async_copy(gx_sh.at[sidx], gxs, sem),
            pltpu.async_copy(gy_sh.at[sidx], gys, sem),
            pltpu.async_copy(gz_sh.at[sidx], gzs, sem),
            pltpu.async_copy(gx_sh.at[didx], gxd, sem),
            pltpu.async_copy(gy_sh.at[didx], gyd, sem),
            pltpu.async_copy(gz_sh.at[didx], gzd, sem),
        ]
        for d in range(DPL):
            descs.append(pltpu.async_copy(feat_sh[d].at[sidx], fpl[d], sem))
        for dd in descs:
            dd.wait()

        def vstep(l, vcarry):
            s = pl.ds(l * 16, 16)
            dx = gxs[s] - gxd[s]
            dy = gys[s] - gyd[s]
            dz = gzs[s] - gzd[s]
            d2b[s] = dx * dx + dy * dy + dz * dz
            return vcarry

        lax.fori_loop(0, CE // 16, vstep, 0)
        pltpu.sync_copy(d2b, d2_hbm.at[pl.ds(eoff, CE)])
        for d in range(DPL):
            pltpu.sync_copy(fpl[d], fT_hbm.at[d, pl.ds(eoff, CE)])
        return carry

    lax.fori_loop(0, CH_G, chunk, 0)


@functools.partial(
    pl.kernel,
    out_type=jax.ShapeDtypeStruct((NC, N_TAB), jnp.float32),
    mesh=_mesh,
    scratch_types=[
        pltpu.VMEM((CS,), jnp.int32),
        pltpu.VMEM((CS,), jnp.float32),
        pltpu.VMEM((N_SLICE,), jnp.float32),
        pltpu.VMEM_SHARED((N_TAB,), jnp.float32),
        pltpu.SemaphoreType.DMA,
    ],
)
def _sc_scatter(dst_hbm, msg_hbm, part_hbm, didx, mv, zbuf, acc, sem):
    cid = lax.axis_index("c")
    sid = lax.axis_index("s")
    wid = sid * NC + cid

    def zstep(i, carry):
        zbuf[pl.ds(i * 16, 16)] = jnp.zeros((16,), jnp.float32)
        return carry
    lax.fori_loop(0, N_SLICE // 16, zstep, 0)
    pltpu.sync_copy(zbuf, acc.at[pl.ds(sid * N_SLICE, N_SLICE)])
    plsc.subcore_barrier()

    e0 = wid * E_PW

    def chunk(t, carry):
        eoff = e0 + t * CS
        pltpu.sync_copy(dst_hbm.at[pl.ds(eoff, CS)], didx)
        pltpu.sync_copy(msg_hbm.at[pl.ds(eoff, CS)], mv)
        pltpu.sync_copy(mv, acc.at[didx], add=True)
        return carry

    lax.fori_loop(0, CH_S, chunk, 0)
    plsc.subcore_barrier()
    pltpu.sync_copy(acc.at[pl.ds(sid * N_SLICE, N_SLICE)],
                    part_hbm.at[cid, pl.ds(sid * N_SLICE, N_SLICE)])


def _make_tc_edge(k_off):
    def _tc_edge_body(d2r, ftr, w1t, b1c, w2t, b2c, out):
        r = jnp.sqrt(d2r[...][0] + 1e-9)
        c8 = lax.broadcasted_iota(jnp.int32, (8, 1), 0).astype(jnp.float32) * STEP
        d = (r - c8) * (1.0 / STEP)
        cosd = jnp.cos((0.5 * np.pi) * d)
        basis = jnp.where(jnp.abs(d) < 1.0, cosd * cosd, 0.0)
        h = jnp.dot(w1t[...], basis, preferred_element_type=jnp.float32) + b1c[...]
        h = jnp.maximum(h, 0.0)
        w = jnp.dot(w2t[...], h, preferred_element_type=jnp.float32) + b2c[...]
        msg = jnp.sum(w[0:DPL, :] * ftr[...], axis=0, keepdims=True)
        eglob = (lax.broadcasted_iota(jnp.int32, (1, EB), 1)
                 + k_off + pl.program_id(0) * EB)
        res = jnp.where(eglob < N_EDGES, msg * INV_NORM, 0.0)
        out[...] = res.reshape(1, 1, EB)

    return pl.pallas_call(
        _tc_edge_body,
        grid=(NBLK_K,),
        in_specs=[
            pl.BlockSpec((1, 1, EB), lambda i: (i, 0, 0)),
            pl.BlockSpec((DPL, EB), lambda i: (0, i)),
            pl.BlockSpec((128, 8), lambda i: (0, 0)),
            pl.BlockSpec((128, 1), lambda i: (0, 0)),
            pl.BlockSpec((32, 128), lambda i: (0, 0)),
            pl.BlockSpec((32, 1), lambda i: (0, 0)),
        ],
        out_specs=pl.BlockSpec((1, 1, EB), lambda i: (i, 0, 0)),
        out_shape=jax.ShapeDtypeStruct((NBLK_K, 1, EB), jnp.float32),
    )


_tc_edges = [_make_tc_edge(k * EK) for k in range(K_CH)]


def _tc_readout_body(parts, w1, b1, w2, b2, w3, b3, out):
    feat = jnp.sum(parts[...], axis=0, keepdims=True)
    h1 = jnp.dot(feat, w1[...], preferred_element_type=jnp.float32) + b1[...]
    h1 = jnp.maximum(h1, 0.0)
    h2 = jnp.dot(h1, w2[...], preferred_element_type=jnp.float32) + b2[...]
    h2 = jnp.maximum(h2, 0.0)
    out[...] = jnp.dot(h2, w3[...], preferred_element_type=jnp.float32) + b3[...]


_tc_readout = pl.pallas_call(
    _tc_readout_body,
    out_shape=jax.ShapeDtypeStruct((1, 8), jnp.float32),
)


def kernel(x, features, geometry, edge_index, W1, b1, W2, b2,
           fc1_W, fc1_b, fc2_W, fc2_b, fc3_W, fc3_b):
    n = features.shape[0]
    e = edge_index.shape[1]

    src = jnp.pad(edge_index[0], (0, E_PAD - e))
    dst = jnp.pad(edge_index[1], (0, E_PAD - e))
    gx = jnp.pad(geometry[:, 0], (0, N_TAB - n))
    gy = jnp.pad(geometry[:, 1], (0, N_TAB - n))
    gz = jnp.pad(geometry[:, 2], (0, N_TAB - n))
    featT = jnp.pad(features.T, ((0, DPL - features.shape[1]), (0, N_TAB - n)))

    w1t = jnp.pad(W1, ((0, 8 - W1.shape[0]), (0, 128 - W1.shape[1]))).T
    b1c = jnp.pad(b1, (0, 128 - b1.shape[0])).reshape(128, 1)
    w2t = jnp.pad(W2, ((0, 128 - W2.shape[0]), (0, 32 - W2.shape[1]))).T
    b2c = jnp.pad(b2, (0, 32 - b2.shape[0])).reshape(32, 1)

    fc1_wp = jnp.pad(fc1_W, ((0, N_TAB - n), (0, 32 - fc1_W.shape[1])))
    fc1_bp = jnp.pad(fc1_b, (0, 32 - fc1_b.shape[0])).reshape(1, 32)
    fc2_wp = jnp.pad(fc2_W, ((0, 32 - fc2_W.shape[0]), (0, 16 - fc2_W.shape[1])))
    fc2_bp = jnp.pad(fc2_b, (0, 16 - fc2_b.shape[0])).reshape(1, 16)
    fc3_wp = jnp.pad(fc3_W, ((0, 16 - fc3_W.shape[0]), (0, 8 - fc3_W.shape[1])))
    fc3_bp = jnp.pad(fc3_b, (0, 8 - fc3_b.shape[0])).reshape(1, 8)

    parts = []
    for k in range(K_CH):
        sl = slice(k * EK, (k + 1) * EK)
        d2, fT = _sc_gather(src[sl], dst[sl], gx, gy, gz, featT)
        msg = _tc_edges[k](d2.reshape(NBLK_K, 1, EB), fT, w1t, b1c, w2t, b2c)
        parts.append(_sc_scatter(dst[sl], msg.reshape(EK)))
    parts_all = jnp.concatenate(parts, axis=0)
    out = _tc_readout(parts_all, fc1_wp, fc1_bp, fc2_wp, fc2_bp,
                      fc3_wp, fc3_bp)
    return out[0, 0:1]
